# Initial kernel scaffold; baseline (speedup 1.0000x reference)
import jax
import jax.numpy as jnp
from jax.experimental import pallas as pl


def kernel(x, Wg, bg, We, be):
    raise NotImplementedError("write your pallas kernel here")


# speedup vs baseline: 2.2313x; 2.2313x over previous
import functools

import jax
import jax.numpy as jnp
from jax.experimental import pallas as pl
from jax.experimental.pallas import tpu as pltpu

EXPERTS = 8
TILE_T = 512


def _moe_body(x_ref, wg_ref, bg_ref, we_ref, be_ref, out_ref, logits_ref):
    x = x_ref[...]

    logits = jax.lax.dot_general(
        x, wg_ref[...],
        dimension_numbers=(((1,), (1,)), ((), ())),
        preferred_element_type=jnp.float32,
    ) + bg_ref[...]
    logits_ref[...] = logits

    cols = [logits[:, e:e + 1] for e in range(EXPERTS)]
    lmax = cols[0]
    for e in range(1, EXPERTS):
        lmax = jnp.maximum(lmax, cols[e])

    weights = []
    denom = None
    for e in range(EXPERTS):
        rank = None
        for j in range(EXPERTS):
            if j == e:
                continue
            if j < e:
                beats = cols[j] >= cols[e]
            else:
                beats = cols[j] > cols[e]
            b = beats.astype(jnp.float32)
            rank = b if rank is None else rank + b
        sel = rank < 2.0
        w = jnp.where(sel, jnp.exp(cols[e] - lmax), 0.0)
        weights.append(w)
        denom = w if denom is None else denom + w

    inv_denom = 1.0 / denom

    acc = None
    for e in range(EXPERTS):
        y = jax.lax.dot_general(
            x, we_ref[e],
            dimension_numbers=(((1,), (1,)), ((), ())),
            preferred_element_type=jnp.float32,
        ) + be_ref[e:e + 1, :]
        contrib = (weights[e] * inv_denom) * y
        acc = contrib if acc is None else acc + contrib
    out_ref[...] = acc


def kernel(x, Wg, bg, We, be):
    B, S, H = x.shape
    h = x.reshape(-1, H)
    T = h.shape[0]
    E = Wg.shape[0]
    bg2 = bg.reshape(1, E)

    grid = (T // TILE_T,)
    out, logits = pl.pallas_call(
        _moe_body,
        grid=grid,
        in_specs=[
            pl.BlockSpec((TILE_T, H), lambda i: (i, 0)),
            pl.BlockSpec((E, H), lambda i: (0, 0)),
            pl.BlockSpec((1, E), lambda i: (0, 0)),
            pl.BlockSpec((E, H, H), lambda i: (0, 0, 0)),
            pl.BlockSpec((E, H), lambda i: (0, 0)),
        ],
        out_specs=[
            pl.BlockSpec((TILE_T, H), lambda i: (i, 0)),
            pl.BlockSpec((TILE_T, E), lambda i: (i, 0)),
        ],
        out_shape=[
            jax.ShapeDtypeStruct((T, H), jnp.float32),
            jax.ShapeDtypeStruct((T, E), jnp.float32),
        ],
    )(h, Wg, bg2, We, be)

    return out.reshape(B, S, H), logits
